# Initial kernel scaffold; baseline (speedup 1.0000x reference)
import jax
import jax.numpy as jnp
from jax.experimental import pallas as pl


def kernel(x, edge_index, edge_features, W, b):
    raise NotImplementedError("write your pallas kernel here")


# speedup vs baseline: 8.1631x; 8.1631x over previous
import functools

import jax
import jax.numpy as jnp
from jax import lax
from jax.experimental import pallas as pl
from jax.experimental.pallas import tpu as pltpu
from jax.experimental.pallas import tpu_sc as plsc

N_NODES = 10000
N_EDGES = 320000
D = 128

NC = 2
NS = 16
L = 16
NW = NC * NS

K = 80
EPW = N_EDGES // NW
N_CHUNKS = EPW // K
NSLAB = 5
CPS = N_CHUNKS // NSLAB
GPS = CPS * K // L
NPAD = 10240
RPT = NPAD // NS
RED = NPAD // NS


def _sc_aggregate(x, src4, dst4, dgrp4, zrows, zflat):
    mesh = plsc.VectorSubcoreMesh(core_axis_name="c", subcore_axis_name="s")

    @functools.partial(
        pl.kernel,
        mesh=mesh,
        compiler_params=pltpu.CompilerParams(needs_layout_passes=False),
        out_type=[
            jax.ShapeDtypeStruct((NC, NPAD, D), jnp.float32),
            jax.ShapeDtypeStruct((NC, NS, NPAD), jnp.float32),
        ],
        scratch_types=[
            pltpu.VMEM((CPS, K), jnp.int32),
            pltpu.VMEM((CPS, K), jnp.int32),
            pltpu.VMEM((GPS, L), jnp.int32),
            pltpu.VMEM((K, D), jnp.float32),
            pltpu.VMEM((NPAD,), jnp.float32),
            pltpu.VMEM_SHARED((NPAD, D), jnp.float32),
            pltpu.SemaphoreType.DMA,
        ],
    )
    def agg(x_hbm, src_hbm, dst_hbm, dgrp_hbm, zrows_hbm, zflat_hbm,
            acc_out, cnt_out,
            sidx_v, didx_v, dgrp_v, rows_v, cnt_t,
            acc_sh, sem):
        cid = lax.axis_index("c")
        sid = lax.axis_index("s")
        wid = sid * NC + cid
        r0 = sid * RPT
        pltpu.sync_copy(zrows_hbm, rows_v)
        pltpu.sync_copy(zflat_hbm, cnt_t)

        @pl.loop(0, RPT // K)
        def zinit(t):
            pltpu.sync_copy(rows_v, acc_sh.at[pl.ds(r0 + t * K, K)])

        plsc.subcore_barrier()

        @pl.loop(0, NSLAB)
        def slab(s):
            pltpu.sync_copy(src_hbm.at[wid, s], sidx_v)
            pltpu.sync_copy(dst_hbm.at[wid, s], didx_v)
            pltpu.sync_copy(dgrp_hbm.at[wid, s], dgrp_v)

            @pl.loop(0, CPS)
            def chunk(j):
                pltpu.async_copy(x_hbm.at[sidx_v.at[j]], rows_v, sem).wait()
                pltpu.sync_copy(rows_v, acc_sh.at[didx_v.at[j]], add=True)

            @pl.loop(0, GPS)
            def grp(g):
                d = dgrp_v[g]
                c, m = plsc.scan_count(d)
                plsc.addupdate_scatter(cnt_t, [d], c.astype(jnp.float32),
                                       mask=m)

        plsc.subcore_barrier()
        pltpu.sync_copy(cnt_t, cnt_out.at[cid, sid])

        @pl.loop(0, RPT // K)
        def export(t):
            off = r0 + t * K
            pltpu.sync_copy(acc_sh.at[pl.ds(off, K)], rows_v)
            pltpu.sync_copy(rows_v, acc_out.at[cid, pl.ds(off, K)])

    return agg(x, src4, dst4, dgrp4, zrows, zflat)


BN = 2560


def _tc_finalize(acc, cnt2, wt, b2):
    def body(acc_ref, cnt_ref, wt_ref, b_ref, o_ref):
        i = pl.program_id(0)
        a = acc_ref[0] + acc_ref[1]
        rb = BN // 128
        craw = cnt_ref[0, pl.ds(i * rb, rb), :]
        for k in range(1, NC * NS):
            craw = craw + cnt_ref[k, pl.ds(i * rb, rb), :]
        nl_r = lax.broadcasted_iota(jnp.int32, (BN, rb), 0)
        kk = lax.broadcasted_iota(jnp.int32, (BN, rb), 1)
        sel_row = (kk == nl_r // 128).astype(jnp.float32)
        nl_c = lax.broadcasted_iota(jnp.int32, (BN, D), 0)
        ll = lax.broadcasted_iota(jnp.int32, (BN, D), 1)
        sel_col = (ll == nl_c % 128).astype(jnp.float32)
        crows = jnp.dot(sel_row, craw, preferred_element_type=jnp.float32)
        c = jnp.sum(crows * sel_col, axis=1, keepdims=True)
        m = jnp.dot(a, wt_ref[...], preferred_element_type=jnp.float32)
        m = (m + c * b_ref[...]) / jnp.maximum(c, 1.0)
        o_ref[...] = jnp.maximum(m, 0.0)

    return pl.pallas_call(
        body,
        grid=(NPAD // BN,),
        in_specs=[
            pl.BlockSpec((NC, BN, D), lambda i: (0, i, 0)),
            pl.BlockSpec((NC * NS, NPAD // 128, 128), lambda i: (0, 0, 0)),
            pl.BlockSpec((D, D), lambda i: (0, 0)),
            pl.BlockSpec((1, D), lambda i: (0, 0)),
        ],
        out_specs=pl.BlockSpec((BN, D), lambda i: (i, 0)),
        out_shape=jax.ShapeDtypeStruct((N_NODES, D), jnp.float32),
    )(acc, cnt2, wt, b2)


def kernel(x, edge_index, edge_features, W, b):
    del edge_features
    src4 = edge_index[0].reshape(NW, NSLAB, CPS, K)
    dst4 = edge_index[1].reshape(NW, NSLAB, CPS, K)
    dgrp4 = edge_index[1].reshape(NW, NSLAB, GPS, L)
    zrows = jnp.zeros((K, D), jnp.float32)
    zflat = jnp.zeros((NPAD,), jnp.float32)
    acc, cnt = _sc_aggregate(x, src4, dst4, dgrp4, zrows, zflat)
    cnt2 = cnt.reshape(NC * NS, NPAD // 128, 128)
    return _tc_finalize(acc, cnt2, W.T, b.reshape(1, D))
